# baseline (speedup 1.0000x reference)
import functools

import numpy as np

import jax
import jax.numpy as jnp
from jax import lax
from jax.experimental import pallas as pl
from jax.experimental.pallas import tpu as pltpu
from jax.experimental.pallas import tpu_sc as plsc

P = 7
RSCALE = 1.2
BSCALE = 1.4
NUM_SC = 2
NUM_SUBCORES = 16
NUM_WORKERS = NUM_SC * NUM_SUBCORES


def _integral_body(x_ref, u_ref, bd_ref, out_ref):
    x = x_ref[0, 0]
    s1 = lax.dot_general(x, u_ref[...], (((1,), (0,)), ((), ())),
                         preferred_element_type=jnp.float32,
                         precision=lax.Precision.HIGHEST)
    s = lax.dot_general(bd_ref[...], s1, (((1,), (0,)), ((), ())),
                        preferred_element_type=jnp.float32,
                        precision=lax.Precision.HIGHEST)
    out_ref[0] = s


def _bins_body(rois_ref, sscale_ref, ia_ref, ib_ref, ic_ref, id_ref, sc_ref,
               h_size, w_size, d_size):
    r = rois_ref[...]
    s = sscale_ref[0, 0]
    bidx = r[:, 0:1].astype(jnp.int32)
    x1 = r[:, 1:2] * s
    y1 = r[:, 2:3] * s
    x2 = r[:, 3:4] * s
    y2 = r[:, 4:5] * s
    roi_w = jnp.maximum(x2 - x1, 0.1)
    roi_h = jnp.maximum(y2 - y1, 0.1)
    cx = 0.5 * (x1 + x2)
    cy = 0.5 * (y1 + y2)
    w_s = roi_w * RSCALE
    h_s = roi_h * RSCALE
    x1s = cx - 0.5 * w_s
    y1s = cy - 0.5 * h_s
    bin_w = w_s / P
    bin_h = h_s / P
    pc = lax.broadcasted_iota(jnp.int32, (1, P * P), 1)
    ph = (pc // P).astype(jnp.float32)
    pw = (pc % P).astype(jnp.float32)
    cyb = y1s + (ph + 0.5) * bin_h
    cxb = x1s + (pw + 0.5) * bin_w
    half_h = 0.5 * BSCALE * bin_h
    half_w = 0.5 * BSCALE * bin_w
    hs = jnp.clip(jnp.floor(cyb - half_h), 0.0, float(h_size))
    he = jnp.clip(jnp.ceil(cyb + half_h), 0.0, float(h_size))
    ws = jnp.clip(jnp.floor(cxb - half_w), 0.0, float(w_size))
    we = jnp.clip(jnp.ceil(cxb + half_w), 0.0, float(w_size))
    cnt = (he - hs) * (we - ws)
    sc_ref[...] = jnp.where(cnt > 0.0, 1.0 / jnp.maximum(cnt, 1.0), 0.0)
    hs_i = hs.astype(jnp.int32)
    he_i = he.astype(jnp.int32)
    ws_i = ws.astype(jnp.int32)
    we_i = we.astype(jnp.int32)
    wj = w_size + 1
    base = (bidx * (P * P) + pc) * ((h_size + 1) * wj)
    ia_ref[...] = base + he_i * wj + we_i
    ib_ref[...] = base + hs_i * wj + we_i
    ic_ref[...] = base + he_i * wj + ws_i
    id_ref[...] = base + hs_i * wj + ws_i


def _make_sc_gather(n_bins, n_rows):
    bins_per_worker = n_bins // NUM_WORKERS
    chunk = 16
    n_chunks = bins_per_worker // chunk
    mesh = plsc.VectorSubcoreMesh(core_axis_name="c", subcore_axis_name="s")

    @functools.partial(
        pl.kernel, mesh=mesh,
        out_type=jax.ShapeDtypeStruct((n_bins, 16), jnp.float32),
        scratch_types=[
            pltpu.VMEM((4 * chunk,), jnp.int32),
            pltpu.VMEM((4 * chunk, 16), jnp.float32),
            pltpu.VMEM((chunk, 16), jnp.float32),
            pltpu.VMEM((chunk, 16), jnp.float32),
            pltpu.SemaphoreType.DMA,
        ],
        compiler_params=pltpu.CompilerParams(use_tc_tiling_on_sc=False),
    )
    def sc_gather(idx_hbm, scale_hbm, table_hbm, out_hbm,
                  idx_v, rows_v, scale_v, out_v, sem):
        wid = lax.axis_index("s") * NUM_SC + lax.axis_index("c")

        def body(k, carry):
            bin_base = wid * bins_per_worker + k * chunk
            pltpu.sync_copy(idx_hbm.at[pl.ds(bin_base * 4, 4 * chunk)], idx_v)
            pltpu.async_copy(table_hbm.at[idx_v], rows_v, sem).wait()
            pltpu.sync_copy(scale_hbm.at[pl.ds(bin_base, chunk)], scale_v)
            for i in range(chunk):
                acc = (rows_v[4 * i, :] - rows_v[4 * i + 1, :]
                       - rows_v[4 * i + 2, :] + rows_v[4 * i + 3, :])
                out_v[i, :] = acc * scale_v[i, :]
            pltpu.sync_copy(out_v, out_hbm.at[pl.ds(bin_base, chunk)])
            return carry

        lax.fori_loop(0, n_chunks, body, 0)

    return sc_gather


def kernel(rois, features, stride):
    B, C, H, W = features.shape
    D = C // (P * P)
    N = rois.shape[0]
    n_bins = N * P * P
    n_rows = B * P * P * (H + 1) * (W + 1)

    x_t = (features.reshape(B, D, P * P, H, W)
           .transpose(0, 2, 1, 3, 4)
           .reshape(B, P * P, D * H, W))

    u_mat = (np.arange(W)[:, None] < np.arange(W + 1)[None, :]).astype(np.float32)
    rr = np.arange(D * (H + 1))
    cc = np.arange(D * H)
    bd_mat = ((rr[:, None] // (H + 1) == cc[None, :] // H)
              & (cc[None, :] % H < rr[:, None] % (H + 1))).astype(np.float32)

    table_dmaj = pl.pallas_call(
        _integral_body,
        grid=(B * P * P,),
        in_specs=[
            pl.BlockSpec((1, 1, D * H, W),
                         lambda g: (g // (P * P), g % (P * P), 0, 0)),
            pl.BlockSpec((W, W + 1), lambda g: (0, 0)),
            pl.BlockSpec((D * (H + 1), D * H), lambda g: (0, 0)),
        ],
        out_specs=pl.BlockSpec((1, D * (H + 1), W + 1), lambda g: (g, 0, 0)),
        out_shape=jax.ShapeDtypeStruct((B * P * P, D * (H + 1), W + 1),
                                       jnp.float32),
    )(x_t, jnp.asarray(u_mat), jnp.asarray(bd_mat))

    table = (table_dmaj.reshape(B * P * P, D, H + 1, W + 1)
             .transpose(0, 2, 3, 1))
    table = jnp.pad(table, ((0, 0), (0, 0), (0, 0), (0, 16 - D)))
    table = table.reshape(n_rows, 16)

    sscale = (1.0 / stride) * jnp.ones((1, 1), jnp.float32)
    bins_body = functools.partial(_bins_body, h_size=H, w_size=W, d_size=D)
    ia, ib, ic, idd, scale = pl.pallas_call(
        bins_body,
        out_shape=[
            jax.ShapeDtypeStruct((N, P * P), jnp.int32),
            jax.ShapeDtypeStruct((N, P * P), jnp.int32),
            jax.ShapeDtypeStruct((N, P * P), jnp.int32),
            jax.ShapeDtypeStruct((N, P * P), jnp.int32),
            jax.ShapeDtypeStruct((N, P * P), jnp.float32),
        ],
    )(rois.astype(jnp.float32), sscale)

    idx_flat = jnp.stack([ia, ib, ic, idd], axis=-1).reshape(n_bins * 4)
    scale16 = jnp.broadcast_to(scale.reshape(n_bins, 1), (n_bins, 16))

    out_rows = _make_sc_gather(n_bins, n_rows)(idx_flat, scale16, table)

    return (out_rows[:, :D].reshape(N, P * P, D)
            .transpose(0, 2, 1).reshape(N, D, P, P))
